# baseline (speedup 1.0000x reference)
import jax
import jax.numpy as jnp
from jax import lax
from jax.experimental import pallas as pl
from jax.experimental.pallas import tpu as pltpu
from jax.experimental.pallas import tpu_sc as plsc

_NC, _NS = 2, 16
_NW = _NC * _NS

_BREAKS = ((11, 1), (24, 1), (26, 2), (40, 1), (60, 1), (61, 2), (62, 1),
           (73, 1))


def _remap_scalar(cp):
    c = cp + 1
    for b, inc in _BREAKS:
        c = c + inc * (cp >= b).astype(jnp.int32)
    return c


def _sc_body(pred_in, enc_in, aux_in, pred_out, enc_out, aux_out, sem):
    wid = lax.axis_index("s") * _NC + lax.axis_index("c")

    n_issued = 0

    for src, dst in ((pred_in, pred_out), (enc_in, enc_out)):
        n_mine = (80 - wid + _NW - 1) // _NW

        def plane_body(i, carry, src=src, dst=dst):
            cp = wid + i * _NW
            pltpu.async_copy(src.at[_remap_scalar(cp)], dst.at[cp], sem)
            return carry

        lax.fori_loop(0, n_mine, plane_body, 0)
        n_issued = n_issued + n_mine

    def aux_body(i, carry):
        u = wid + i * _NW
        a = u // 80
        cp = u % 80
        pltpu.async_copy(aux_in.at[a, _remap_scalar(cp)], aux_out.at[a, cp],
                         sem)
        return carry

    lax.fori_loop(0, 480 // _NW, aux_body, 0)
    n_issued = n_issued + 480 // _NW

    def drain_body(i, carry):
        pltpu.make_async_copy(pred_in.at[0], pred_out.at[0], sem).wait()
        return carry

    lax.fori_loop(0, n_issued, drain_body, 0)


@jax.jit
def kernel(pred_logits, enc_pred_logits, aux_pred_logits):
    mesh = plsc.VectorSubcoreMesh(core_axis_name="c", subcore_axis_name="s",
                                  num_cores=_NC, num_subcores=_NS)
    run = pl.kernel(
        _sc_body,
        out_type=(
            jax.ShapeDtypeStruct((80, 16, 900), jnp.float32),
            jax.ShapeDtypeStruct((80, 16, 900), jnp.float32),
            jax.ShapeDtypeStruct((6, 80, 16, 900), jnp.float32),
        ),
        mesh=mesh,
        scratch_types=[
            pltpu.SemaphoreType.DMA,
        ],
        compiler_params=pltpu.CompilerParams(needs_layout_passes=False),
    )
    out_t, enc_t, aux_t = run(pred_logits.transpose(2, 0, 1),
                              enc_pred_logits.transpose(2, 0, 1),
                              aux_pred_logits.transpose(0, 3, 1, 2))
    return (out_t.transpose(1, 2, 0),
            enc_t.transpose(1, 2, 0),
            aux_t.transpose(0, 2, 3, 1))


# speedup vs baseline: 15.1012x; 15.1012x over previous
import jax
import jax.numpy as jnp
from jax import lax
from jax.experimental import pallas as pl
from jax.experimental.pallas import tpu as pltpu
from jax.experimental.pallas import tpu_sc as plsc

_NC, _NS = 2, 16
_NW = _NC * _NS

_BREAKS = ((11, 1), (24, 1), (26, 2), (40, 1), (60, 1), (61, 2), (62, 1),
           (73, 1))


def _remap_scalar(cp):
    c = cp + 1
    for b, inc in _BREAKS:
        c = c + inc * (cp >= b).astype(jnp.int32)
    return c


def _sc_body(pred_in, enc_in, aux_in, pred_out, enc_out, aux_out,
             buf0, buf1, si0, si1, so0, so1):
    wid = lax.axis_index("s") * _NC + lax.axis_index("c")
    bufs = ((buf0, si0, so0), (buf1, si1, so1))

    def run_tensor(src_plane, dst_plane, n_mine):
        for b, (buf, si, so) in enumerate(bufs):
            @pl.when(n_mine > b)
            def _():
                pltpu.async_copy(src_plane(b), buf, si)

        def pair_body(k, carry):
            for b, (buf, si, so) in enumerate(bufs):
                i = 2 * k + b

                @pl.when(i < n_mine)
                def _(i=i, buf=buf, si=si, so=so):
                    pltpu.make_async_copy(src_plane(i), buf, si).wait()
                    pltpu.async_copy(buf, dst_plane(i), so)
                    pltpu.make_async_copy(buf, dst_plane(i), so).wait()

                    @pl.when(i + 2 < n_mine)
                    def _():
                        pltpu.async_copy(src_plane(i + 2), buf, si)
            return carry

        lax.fori_loop(0, (n_mine + 1) // 2, pair_body, 0)

    for src, dst in ((pred_in, pred_out), (enc_in, enc_out)):
        n_mine = (80 - wid + _NW - 1) // _NW
        run_tensor(
            lambda i, src=src: src.at[_remap_scalar(wid + i * _NW)],
            lambda i, dst=dst: dst.at[wid + i * _NW],
            n_mine)

    def aux_src(i):
        u = wid + i * _NW
        return aux_in.at[u // 80, _remap_scalar(u % 80)]

    def aux_dst(i):
        u = wid + i * _NW
        return aux_out.at[u // 80, u % 80]

    run_tensor(aux_src, aux_dst, 480 // _NW)


@jax.jit
def kernel(pred_logits, enc_pred_logits, aux_pred_logits):
    mesh = plsc.VectorSubcoreMesh(core_axis_name="c", subcore_axis_name="s",
                                  num_cores=_NC, num_subcores=_NS)
    run = pl.kernel(
        _sc_body,
        out_type=(
            jax.ShapeDtypeStruct((80, 16, 900), jnp.float32),
            jax.ShapeDtypeStruct((80, 16, 900), jnp.float32),
            jax.ShapeDtypeStruct((6, 80, 16, 900), jnp.float32),
        ),
        mesh=mesh,
        scratch_types=[
            pltpu.VMEM((16, 900), jnp.float32),
            pltpu.VMEM((16, 900), jnp.float32),
            pltpu.SemaphoreType.DMA,
            pltpu.SemaphoreType.DMA,
            pltpu.SemaphoreType.DMA,
            pltpu.SemaphoreType.DMA,
        ],
        compiler_params=pltpu.CompilerParams(needs_layout_passes=False),
    )
    out_t, enc_t, aux_t = run(pred_logits.transpose(2, 0, 1),
                              enc_pred_logits.transpose(2, 0, 1),
                              aux_pred_logits.transpose(0, 3, 1, 2))
    return (out_t.transpose(1, 2, 0),
            enc_t.transpose(1, 2, 0),
            aux_t.transpose(0, 2, 3, 1))


# speedup vs baseline: 20.0006x; 1.3244x over previous
import jax
import jax.numpy as jnp
from jax import lax
from jax.experimental import pallas as pl
from jax.experimental.pallas import tpu as pltpu
from jax.experimental.pallas import tpu_sc as plsc

_NC, _NS = 2, 16
_NW = _NC * _NS

_RUNS_H = (
    ((1, 0, 11), (13, 11, 13), (27, 24, 2), (31, 26, 14)),
    ((46, 0, 20), (67, 20, 1), (70, 21, 1), (72, 22, 11), (84, 33, 7)),
)


def _sc_body(pred_in, enc_in, aux_in, pred_out, enc_out, aux_out,
             buf_a, buf_b, si, so):
    wid = lax.axis_index("s") * _NC + lax.axis_index("c")

    def do_unit(six, dix, bg, q0, runs, buf, qlen, cbase):
        b8 = bg * 8
        for s, d, ln in runs:
            pltpu.async_copy(six(s, ln, b8, q0, qlen),
                             buf.at[pl.ds(d, ln)], si)
        for s, d, ln in runs:
            pltpu.make_async_copy(six(s, ln, b8, q0, qlen),
                                  buf.at[pl.ds(d, ln)], si).wait()
        for b in range(8):
            pltpu.async_copy(buf.at[:, b, :], dix(b8 + b, cbase, q0, qlen),
                             so)
        for b in range(8):
            pltpu.make_async_copy(buf.at[:, b, :],
                                  dix(b8 + b, cbase, q0, qlen), so).wait()

    def make_ix(src, dst, a=None):
        if a is None:
            six = lambda s, ln, b8, q0, ql: src.at[
                pl.ds(s, ln), pl.ds(b8, 8), pl.ds(q0, ql)]
            dix = lambda bb, cb, q0, ql: dst.at[
                bb, pl.ds(cb, 40), pl.ds(q0, ql)]
        else:
            six = lambda s, ln, b8, q0, ql: src.at[
                a, pl.ds(s, ln), pl.ds(b8, 8), pl.ds(q0, ql)]
            dix = lambda bb, cb, q0, ql: dst.at[
                a, bb, pl.ds(cb, 40), pl.ds(q0, ql)]
        return six, dix

    def dispatch_slab(slab, fn):
        def on_pred(_):
            fn(*make_ix(pred_in, pred_out))
            return 0

        def on_enc(_):
            fn(*make_ix(enc_in, enc_out))
            return 0

        def on_aux(_):
            fn(*make_ix(aux_in, aux_out, slab - 2))
            return 0

        return lax.switch(jnp.minimum(slab, 2), (on_pred, on_enc, on_aux), 0)

    def do_halves(six, dix, bg, q0, buf, qlen, ch):
        def h0(_):
            do_unit(six, dix, bg, q0, _RUNS_H[0], buf, qlen, 0)
            return 0

        def h1(_):
            do_unit(six, dix, bg, q0, _RUNS_H[1], buf, qlen, 40)
            return 0

        return lax.cond(ch == 0, h0, h1, 0)

    def unit_body(i, carry):
        g = wid + i * _NW

        def main(gg):
            slab = gg // 24
            r = gg % 24
            bg = r // 12
            qt = (r % 12) // 2
            ch = r % 2
            return dispatch_slab(
                slab,
                lambda six, dix: do_halves(six, dix, bg, qt * 128, buf_a,
                                           128, ch))

        def last(gg):
            v = gg - 192
            slab = v // 4
            bg = (v % 4) // 2
            ch = v % 2
            return dispatch_slab(
                slab,
                lambda six, dix: do_halves(six, dix, bg, 768, buf_b, 132,
                                           ch))

        lax.cond(g < 192, main, last, g)
        return carry

    lax.fori_loop(0, 7, unit_body, 0)


@jax.jit
def kernel(pred_logits, enc_pred_logits, aux_pred_logits):
    mesh = plsc.VectorSubcoreMesh(core_axis_name="c", subcore_axis_name="s",
                                  num_cores=_NC, num_subcores=_NS)
    run = pl.kernel(
        _sc_body,
        out_type=(
            jax.ShapeDtypeStruct((16, 80, 900), jnp.float32),
            jax.ShapeDtypeStruct((16, 80, 900), jnp.float32),
            jax.ShapeDtypeStruct((6, 16, 80, 900), jnp.float32),
        ),
        mesh=mesh,
        scratch_types=[
            pltpu.VMEM((40, 8, 128), jnp.float32),
            pltpu.VMEM((40, 8, 132), jnp.float32),
            pltpu.SemaphoreType.DMA,
            pltpu.SemaphoreType.DMA,
        ],
        compiler_params=pltpu.CompilerParams(needs_layout_passes=False),
    )
    out_t, enc_t, aux_t = run(pred_logits.transpose(2, 0, 1),
                              enc_pred_logits.transpose(2, 0, 1),
                              aux_pred_logits.transpose(0, 3, 1, 2))
    return (out_t.transpose(0, 2, 1),
            enc_t.transpose(0, 2, 1),
            aux_t.transpose(0, 1, 3, 2))
